# Initial kernel scaffold; baseline (speedup 1.0000x reference)
import jax
import jax.numpy as jnp
from jax.experimental import pallas as pl


def kernel(x, edge_index, shuffled_index, sample_batch, W1, b1, W2, b2, Wd, bd):
    raise NotImplementedError("write your pallas kernel here")


# speedup vs baseline: 3.3197x; 3.3197x over previous
import functools

import jax
import jax.numpy as jnp
from jax import lax
from jax.experimental import pallas as pl
from jax.experimental.pallas import tpu as pltpu
from jax.experimental.pallas import tpu_sc as plsc

N = 10000
E = 160000
D_IN = 256
HID = 512
OUTF = 256
DECF = 256
BB = 1024
KK = 16

NC = 2
NS = 16
NW = NC * NS
HALF = D_IN // 2
CH = 128
NCHUNK = E // CH
ECPT = -(-NCHUNK // NS)
RW = 200
NRCH = N // RW
RCPT = -(-NRCH // NS)

_SC_MESH = dict(core_axis_name="c", subcore_axis_name="s",
                num_cores=NC, num_subcores=NS)


def _row_chunks(s, fn):
  def body(j, carry):
    cid = s + NS * j

    @pl.when(cid < NRCH)
    def _():
      fn(cid * RW)
    return carry

  lax.fori_loop(0, RCPT, body, 0)


def _seg_loop(c, s, src_h, dst_h, xA_h, xB_h, idx_v, dsti_v, rows_v,
              acc_s, sem):
  def body(j, carry):
    cid = s + NS * j

    @pl.when(cid < NCHUNK)
    def _():
      off = cid * CH
      pltpu.sync_copy(src_h.at[pl.ds(off, CH)], idx_v)
      pltpu.sync_copy(dst_h.at[pl.ds(off, CH)], dsti_v)

      @pl.when(c == 0)
      def _():
        pltpu.async_copy(xA_h.at[idx_v], rows_v, sem).wait()

      @pl.when(c == 1)
      def _():
        pltpu.async_copy(xB_h.at[idx_v], rows_v, sem).wait()

      pltpu.sync_copy(rows_v, acc_s.at[dsti_v], add=True)
    return carry

  lax.fori_loop(0, ECPT, body, 0)


def _sc_segsum(xA, xB, src, dst, zrow, w):
  @functools.partial(
      pl.kernel,
      out_type=(jax.ShapeDtypeStruct((N, w), jnp.float32),
                jax.ShapeDtypeStruct((N, w), jnp.float32)),
      mesh=plsc.VectorSubcoreMesh(**_SC_MESH),
      scratch_types=[
          pltpu.VMEM((CH,), jnp.int32),
          pltpu.VMEM((CH,), jnp.int32),
          pltpu.VMEM((CH, w), jnp.float32),
          pltpu.VMEM_SHARED((N, w), jnp.float32),
          pltpu.SemaphoreType.DMA,
      ],
  )
  def k(xA_h, xB_h, src_h, dst_h, zrow_h, aggA_h, aggB_h,
        idx_v, dsti_v, rows_v, acc_s, sem):
    c = lax.axis_index("c")
    s = lax.axis_index("s")
    _row_chunks(s, lambda off: pltpu.sync_copy(zrow_h, acc_s.at[pl.ds(off, RW)]))
    plsc.subcore_barrier()
    _seg_loop(c, s, src_h, dst_h, xA_h, xB_h, idx_v, dsti_v, rows_v,
              acc_s, sem)
    plsc.subcore_barrier()

    def wb(off):
      @pl.when(c == 0)
      def _():
        pltpu.sync_copy(acc_s.at[pl.ds(off, RW)], aggA_h.at[pl.ds(off, RW)])

      @pl.when(c == 1)
      def _():
        pltpu.sync_copy(acc_s.at[pl.ds(off, RW)], aggB_h.at[pl.ds(off, RW)])

    _row_chunks(s, wb)

  return k(xA, xB, src, dst, zrow)


DCPT = -(-NCHUNK // NW)


def _sc_deg(dst, ones_h, zrow):
  @functools.partial(
      pl.kernel,
      out_type=(jax.ShapeDtypeStruct((N, HALF), jnp.float32),
                jax.ShapeDtypeStruct((N, HALF), jnp.float32)),
      mesh=plsc.VectorSubcoreMesh(**_SC_MESH),
      scratch_types=[
          pltpu.VMEM((CH,), jnp.int32),
          pltpu.VMEM((CH, HALF), jnp.float32),
          pltpu.VMEM_SHARED((N, HALF), jnp.float32),
      ],
  )
  def k(dst_h, ones_hh, zrow_h, degA_h, degB_h, dsti_v, ones_v, acc_s):
    c = lax.axis_index("c")
    s = lax.axis_index("s")
    wid = s * NC + c
    _row_chunks(s, lambda off: pltpu.sync_copy(zrow_h, acc_s.at[pl.ds(off, RW)]))
    pltpu.sync_copy(ones_hh, ones_v)
    plsc.subcore_barrier()

    def body(j, carry):
      cid = wid + NW * j

      @pl.when(cid < NCHUNK)
      def _():
        off = cid * CH
        pltpu.sync_copy(dst_h.at[pl.ds(off, CH)], dsti_v)
        pltpu.sync_copy(ones_v, acc_s.at[dsti_v], add=True)
      return carry

    lax.fori_loop(0, DCPT, body, 0)
    plsc.subcore_barrier()

    def wb(off):
      @pl.when(c == 0)
      def _():
        pltpu.sync_copy(acc_s.at[pl.ds(off, RW)], degA_h.at[pl.ds(off, RW)])

      @pl.when(c == 1)
      def _():
        pltpu.sync_copy(acc_s.at[pl.ds(off, RW)], degB_h.at[pl.ds(off, RW)])

    _row_chunks(s, wb)

  return k(dst, ones_h, zrow)


NG = 32768
GPT = NG // NW
GCH = 128


def _sc_gather(h, idx):
  @functools.partial(
      pl.kernel,
      out_type=jax.ShapeDtypeStruct((NG, OUTF), jnp.float32),
      mesh=plsc.VectorSubcoreMesh(**_SC_MESH),
      scratch_types=[
          pltpu.VMEM((GCH,), jnp.int32),
          pltpu.VMEM((GCH, OUTF), jnp.float32),
          pltpu.SemaphoreType.DMA,
      ],
  )
  def k(h_h, idx_h, out_h, idx_v, rows_v, sem):
    c = lax.axis_index("c")
    s = lax.axis_index("s")
    wid = s * NC + c

    def body(j, carry):
      off = wid * GPT + j * GCH
      pltpu.sync_copy(idx_h.at[pl.ds(off, GCH)], idx_v)
      pltpu.async_copy(h_h.at[idx_v], rows_v, sem).wait()
      pltpu.sync_copy(rows_v, out_h.at[pl.ds(off, GCH)])
      return carry

    lax.fori_loop(0, GPT // GCH, body, 0)

  return k(h, idx)


RB = 400


def _tc_stage1(x, aggA, aggB, degA, degB, W1, b1, W2, b2):
  def body(x_r, aA_r, aB_r, dA_r, dB_r, W1_r, b1_r, W2_r, b2_r, oA_r, oB_r):
    agg = jnp.concatenate([aA_r[...], aB_r[...]], axis=1)
    rec = 1.0 / (dA_r[...][:, 0:1] + dB_r[...][:, 0:1] + 1.0)
    z1 = (agg + x_r[...]) * rec
    h1 = jnp.dot(z1, W1_r[...], preferred_element_type=jnp.float32) + b1_r[...]
    h1 = jnp.maximum(h1, 0.0)
    y2 = jnp.dot(h1, W2_r[...], preferred_element_type=jnp.float32) + b2_r[...]
    oA_r[...] = y2[:, :HALF]
    oB_r[...] = y2[:, HALF:]

  return pl.pallas_call(
      body,
      grid=(N // RB,),
      in_specs=[
          pl.BlockSpec((RB, D_IN), lambda i: (i, 0)),
          pl.BlockSpec((RB, HALF), lambda i: (i, 0)),
          pl.BlockSpec((RB, HALF), lambda i: (i, 0)),
          pl.BlockSpec((RB, HALF), lambda i: (i, 0)),
          pl.BlockSpec((RB, HALF), lambda i: (i, 0)),
          pl.BlockSpec((D_IN, HID), lambda i: (0, 0)),
          pl.BlockSpec((1, HID), lambda i: (0, 0)),
          pl.BlockSpec((HID, OUTF), lambda i: (0, 0)),
          pl.BlockSpec((1, OUTF), lambda i: (0, 0)),
      ],
      out_specs=[
          pl.BlockSpec((RB, HALF), lambda i: (i, 0)),
          pl.BlockSpec((RB, HALF), lambda i: (i, 0)),
      ],
      out_shape=[
          jax.ShapeDtypeStruct((N, HALF), jnp.float32),
          jax.ShapeDtypeStruct((N, HALF), jnp.float32),
      ],
  )(x, aggA, aggB, degA, degB, W1, b1, W2, b2)


def _tc_stage2(y2A, y2B, aggA, aggB, degA, degB):
  def body(yA_r, yB_r, aA_r, aB_r, dA_r, dB_r, h_r):
    rec = 1.0 / (dA_r[...][:, 0:1] + dB_r[...][:, 0:1] + 1.0)
    left = (aA_r[...] + yA_r[...]) * rec
    right = (aB_r[...] + yB_r[...]) * rec
    h_r[...] = jnp.concatenate([left, right], axis=1)

  return pl.pallas_call(
      body,
      grid=(N // RB,),
      in_specs=[pl.BlockSpec((RB, HALF), lambda i: (i, 0))] * 6,
      out_specs=pl.BlockSpec((RB, OUTF), lambda i: (i, 0)),
      out_shape=jax.ShapeDtypeStruct((N, OUTF), jnp.float32),
  )(y2A, y2B, aggA, aggB, degA, degB)


DB = 2048


def _tc_decoder(ha_rows, hb_rows, Wd, bd):
  def body(a_r, b_r, Wd_r, bd_r, o_r):
    ha = jnp.dot(a_r[...], Wd_r[...], preferred_element_type=jnp.float32) + bd_r[...]
    hb = jnp.dot(b_r[...], Wd_r[...], preferred_element_type=jnp.float32) + bd_r[...]
    num = jnp.sum(ha * hb, axis=1, keepdims=True)
    na = jnp.sqrt(jnp.sum(ha * ha, axis=1, keepdims=True))
    nb = jnp.sqrt(jnp.sum(hb * hb, axis=1, keepdims=True))
    o_r[...] = num / jnp.maximum(na * nb, 1e-8)

  return pl.pallas_call(
      body,
      grid=(BB * KK // DB,),
      in_specs=[
          pl.BlockSpec((DB, OUTF), lambda i: (i, 0)),
          pl.BlockSpec((DB, OUTF), lambda i: (i, 0)),
          pl.BlockSpec((OUTF, DECF), lambda i: (0, 0)),
          pl.BlockSpec((1, DECF), lambda i: (0, 0)),
      ],
      out_specs=pl.BlockSpec((DB, 1), lambda i: (i, 0)),
      out_shape=jax.ShapeDtypeStruct((BB * KK, 1), jnp.float32),
  )(ha_rows, hb_rows, Wd, bd)


def kernel(x, edge_index, shuffled_index, sample_batch, W1, b1, W2, b2, Wd, bd):
  src = edge_index[0]
  dst = edge_index[1]
  xA = x[:, :HALF]
  xB = x[:, HALF:]
  zrow = jnp.zeros((RW, HALF), jnp.float32)
  ones_h = jnp.ones((CH, HALF), jnp.float32)

  degA, degB = _sc_deg(dst, ones_h, zrow)
  aggA, aggB = _sc_segsum(xA, xB, src, dst, zrow, HALF)
  y2A, y2B = _tc_stage1(x, aggA, aggB, degA, degB,
                        W1, b1.reshape(1, HID), W2, b2.reshape(1, OUTF))
  agg2A, agg2B = _sc_segsum(y2A, y2B, src, dst, zrow, HALF)
  h = _tc_stage2(y2A, y2B, agg2A, agg2B, degA, degB)

  bidx_rep = jnp.repeat(sample_batch, KK)
  gidx = jnp.concatenate([bidx_rep, shuffled_index.reshape(-1)])
  rows = _sc_gather(h, gidx)
  ha_rows = rows[:BB * KK]
  hb_rows = rows[BB * KK:]
  dec = _tc_decoder(ha_rows, hb_rows, Wd, bd.reshape(1, DECF)).reshape(BB, KK)
  return (h, dec)


# speedup vs baseline: 4.3192x; 1.3011x over previous
import functools

import jax
import jax.numpy as jnp
from jax import lax
from jax.experimental import pallas as pl
from jax.experimental.pallas import tpu as pltpu
from jax.experimental.pallas import tpu_sc as plsc

N = 10000
E = 160000
D_IN = 256
HID = 512
OUTF = 256
DECF = 256
BB = 1024
KK = 16

NC = 2
NS = 16
NW = NC * NS
HALF = D_IN // 2
CH = 128
NCHUNK = E // CH
QPT = -(-NCHUNK // NS)
G = 3
RW = 200
NRCH = N // RW
RCPT = -(-NRCH // NS)

_SC_MESH = dict(core_axis_name="c", subcore_axis_name="s",
                num_cores=NC, num_subcores=NS)


def _row_chunks(s, fn):
  def body(j, carry):
    cid = s + NS * j

    @pl.when(cid < NRCH)
    def _():
      fn(cid * RW)
    return carry

  lax.fori_loop(0, RCPT, body, 0)


def _seg_pipe(s, tab_h, src_h, dst_h, sidx, didx, rows, semi, semj, semg,
              acc_s):
  lo = jnp.minimum(s * QPT, NCHUNK)
  hi = jnp.minimum(lo + QPT, NCHUNK)
  ngroups = (hi - lo) // G

  def group(g, _):
    cid0 = lo + g * G
    di = []
    dj = []
    for b in range(G):
      off = (cid0 + b) * CH
      di.append(pltpu.async_copy(src_h.at[pl.ds(off, CH)], sidx[b],
                                 semi.at[b]))
      dj.append(pltpu.async_copy(dst_h.at[pl.ds(off, CH)], didx[b],
                                 semj.at[b]))
    dg = []
    for b in range(G):
      di[b].wait()
      dg.append(pltpu.async_copy(tab_h.at[sidx[b]], rows[b], semg.at[b]))
    for b in range(G):
      dg[b].wait()
      dj[b].wait()
      pltpu.sync_copy(rows[b], acc_s.at[didx[b]], add=True)
    return _

  lax.fori_loop(0, ngroups, group, 0)

  def tail(tt, _):
    cid = lo + ngroups * G + tt
    off = cid * CH
    pltpu.sync_copy(src_h.at[pl.ds(off, CH)], sidx[0])
    pltpu.sync_copy(dst_h.at[pl.ds(off, CH)], didx[0])
    pltpu.async_copy(tab_h.at[sidx[0]], rows[0], semg.at[0]).wait()
    pltpu.sync_copy(rows[0], acc_s.at[didx[0]], add=True)
    return _

  lax.fori_loop(0, hi - lo - ngroups * G, tail, 0)


def _sc_segsum(xA, xB, src, dst, zrow, w):
  @functools.partial(
      pl.kernel,
      out_type=(jax.ShapeDtypeStruct((N, w), jnp.float32),
                jax.ShapeDtypeStruct((N, w), jnp.float32)),
      mesh=plsc.VectorSubcoreMesh(**_SC_MESH),
      scratch_types=(
          [pltpu.VMEM((CH,), jnp.int32)] * G +
          [pltpu.VMEM((CH,), jnp.int32)] * G +
          [pltpu.VMEM((CH, w), jnp.float32)] * G +
          [pltpu.VMEM_SHARED((N, w), jnp.float32),
           pltpu.SemaphoreType.DMA((G,)),
           pltpu.SemaphoreType.DMA((G,)),
           pltpu.SemaphoreType.DMA((G,))]),
  )
  def k(xA_h, xB_h, src_h, dst_h, zrow_h, aggA_h, aggB_h, *refs):
    sidx = refs[:G]
    didx = refs[G:2 * G]
    rows = refs[2 * G:3 * G]
    acc_s, semi, semj, semg = refs[3 * G:]
    c = lax.axis_index("c")
    s = lax.axis_index("s")
    _row_chunks(s, lambda off: pltpu.sync_copy(zrow_h, acc_s.at[pl.ds(off, RW)]))
    plsc.subcore_barrier()

    @pl.when(c == 0)
    def _():
      _seg_pipe(s, xA_h, src_h, dst_h, sidx, didx, rows, semi, semj, semg,
                acc_s)

    @pl.when(c == 1)
    def _():
      _seg_pipe(s, xB_h, src_h, dst_h, sidx, didx, rows, semi, semj, semg,
                acc_s)

    plsc.subcore_barrier()

    def wb(off):
      @pl.when(c == 0)
      def _():
        pltpu.sync_copy(acc_s.at[pl.ds(off, RW)], aggA_h.at[pl.ds(off, RW)])

      @pl.when(c == 1)
      def _():
        pltpu.sync_copy(acc_s.at[pl.ds(off, RW)], aggB_h.at[pl.ds(off, RW)])

    _row_chunks(s, wb)

  return k(xA, xB, src, dst, zrow)


DCPT = -(-NCHUNK // NW)


def _sc_deg(dst, ones_h, zrow):
  @functools.partial(
      pl.kernel,
      out_type=(jax.ShapeDtypeStruct((N, HALF), jnp.float32),
                jax.ShapeDtypeStruct((N, HALF), jnp.float32)),
      mesh=plsc.VectorSubcoreMesh(**_SC_MESH),
      scratch_types=[
          pltpu.VMEM((CH,), jnp.int32),
          pltpu.VMEM((CH, HALF), jnp.float32),
          pltpu.VMEM_SHARED((N, HALF), jnp.float32),
      ],
  )
  def k(dst_h, ones_hh, zrow_h, degA_h, degB_h, dsti_v, ones_v, acc_s):
    c = lax.axis_index("c")
    s = lax.axis_index("s")
    wid = s * NC + c
    _row_chunks(s, lambda off: pltpu.sync_copy(zrow_h, acc_s.at[pl.ds(off, RW)]))
    pltpu.sync_copy(ones_hh, ones_v)
    plsc.subcore_barrier()

    def body(j, carry):
      cid = wid + NW * j

      @pl.when(cid < NCHUNK)
      def _():
        off = cid * CH
        pltpu.sync_copy(dst_h.at[pl.ds(off, CH)], dsti_v)
        pltpu.sync_copy(ones_v, acc_s.at[dsti_v], add=True)
      return carry

    lax.fori_loop(0, DCPT, body, 0)
    plsc.subcore_barrier()

    def wb(off):
      @pl.when(c == 0)
      def _():
        pltpu.sync_copy(acc_s.at[pl.ds(off, RW)], degA_h.at[pl.ds(off, RW)])

      @pl.when(c == 1)
      def _():
        pltpu.sync_copy(acc_s.at[pl.ds(off, RW)], degB_h.at[pl.ds(off, RW)])

    _row_chunks(s, wb)

  return k(dst, ones_h, zrow)


NG = 32768
GPT = NG // NW
GCH = 128


def _sc_gather(h, idx):
  @functools.partial(
      pl.kernel,
      out_type=jax.ShapeDtypeStruct((NG, OUTF), jnp.float32),
      mesh=plsc.VectorSubcoreMesh(**_SC_MESH),
      scratch_types=[
          pltpu.VMEM((GCH,), jnp.int32),
          pltpu.VMEM((GCH, OUTF), jnp.float32),
          pltpu.SemaphoreType.DMA,
      ],
  )
  def k(h_h, idx_h, out_h, idx_v, rows_v, sem):
    c = lax.axis_index("c")
    s = lax.axis_index("s")
    wid = s * NC + c

    def body(j, carry):
      off = wid * GPT + j * GCH
      pltpu.sync_copy(idx_h.at[pl.ds(off, GCH)], idx_v)
      pltpu.async_copy(h_h.at[idx_v], rows_v, sem).wait()
      pltpu.sync_copy(rows_v, out_h.at[pl.ds(off, GCH)])
      return carry

    lax.fori_loop(0, GPT // GCH, body, 0)

  return k(h, idx)


RB = 400


def _tc_stage1(x, aggA, aggB, degA, degB, W1, b1, W2, b2):
  def body(x_r, aA_r, aB_r, dA_r, dB_r, W1_r, b1_r, W2_r, b2_r, oA_r, oB_r):
    agg = jnp.concatenate([aA_r[...], aB_r[...]], axis=1)
    rec = 1.0 / (dA_r[...][:, 0:1] + dB_r[...][:, 0:1] + 1.0)
    z1 = (agg + x_r[...]) * rec
    h1 = jnp.dot(z1, W1_r[...], preferred_element_type=jnp.float32) + b1_r[...]
    h1 = jnp.maximum(h1, 0.0)
    y2 = jnp.dot(h1, W2_r[...], preferred_element_type=jnp.float32) + b2_r[...]
    oA_r[...] = y2[:, :HALF]
    oB_r[...] = y2[:, HALF:]

  return pl.pallas_call(
      body,
      grid=(N // RB,),
      in_specs=[
          pl.BlockSpec((RB, D_IN), lambda i: (i, 0)),
          pl.BlockSpec((RB, HALF), lambda i: (i, 0)),
          pl.BlockSpec((RB, HALF), lambda i: (i, 0)),
          pl.BlockSpec((RB, HALF), lambda i: (i, 0)),
          pl.BlockSpec((RB, HALF), lambda i: (i, 0)),
          pl.BlockSpec((D_IN, HID), lambda i: (0, 0)),
          pl.BlockSpec((1, HID), lambda i: (0, 0)),
          pl.BlockSpec((HID, OUTF), lambda i: (0, 0)),
          pl.BlockSpec((1, OUTF), lambda i: (0, 0)),
      ],
      out_specs=[
          pl.BlockSpec((RB, HALF), lambda i: (i, 0)),
          pl.BlockSpec((RB, HALF), lambda i: (i, 0)),
      ],
      out_shape=[
          jax.ShapeDtypeStruct((N, HALF), jnp.float32),
          jax.ShapeDtypeStruct((N, HALF), jnp.float32),
      ],
  )(x, aggA, aggB, degA, degB, W1, b1, W2, b2)


def _tc_stage2(y2A, y2B, aggA, aggB, degA, degB):
  def body(yA_r, yB_r, aA_r, aB_r, dA_r, dB_r, h_r):
    rec = 1.0 / (dA_r[...][:, 0:1] + dB_r[...][:, 0:1] + 1.0)
    left = (aA_r[...] + yA_r[...]) * rec
    right = (aB_r[...] + yB_r[...]) * rec
    h_r[...] = jnp.concatenate([left, right], axis=1)

  return pl.pallas_call(
      body,
      grid=(N // RB,),
      in_specs=[pl.BlockSpec((RB, HALF), lambda i: (i, 0))] * 6,
      out_specs=pl.BlockSpec((RB, OUTF), lambda i: (i, 0)),
      out_shape=jax.ShapeDtypeStruct((N, OUTF), jnp.float32),
  )(y2A, y2B, aggA, aggB, degA, degB)


DB = 2048


def _tc_decoder(ha_rows, hb_rows, Wd, bd):
  def body(a_r, b_r, Wd_r, bd_r, o_r):
    ha = jnp.dot(a_r[...], Wd_r[...], preferred_element_type=jnp.float32) + bd_r[...]
    hb = jnp.dot(b_r[...], Wd_r[...], preferred_element_type=jnp.float32) + bd_r[...]
    num = jnp.sum(ha * hb, axis=1, keepdims=True)
    na = jnp.sqrt(jnp.sum(ha * ha, axis=1, keepdims=True))
    nb = jnp.sqrt(jnp.sum(hb * hb, axis=1, keepdims=True))
    o_r[...] = num / jnp.maximum(na * nb, 1e-8)

  return pl.pallas_call(
      body,
      grid=(BB * KK // DB,),
      in_specs=[
          pl.BlockSpec((DB, OUTF), lambda i: (i, 0)),
          pl.BlockSpec((DB, OUTF), lambda i: (i, 0)),
          pl.BlockSpec((OUTF, DECF), lambda i: (0, 0)),
          pl.BlockSpec((1, DECF), lambda i: (0, 0)),
      ],
      out_specs=pl.BlockSpec((DB, 1), lambda i: (i, 0)),
      out_shape=jax.ShapeDtypeStruct((BB * KK, 1), jnp.float32),
  )(ha_rows, hb_rows, Wd, bd)


def kernel(x, edge_index, shuffled_index, sample_batch, W1, b1, W2, b2, Wd, bd):
  src = edge_index[0]
  dst = edge_index[1]
  xA = x[:, :HALF]
  xB = x[:, HALF:]
  zrow = jnp.zeros((RW, HALF), jnp.float32)
  ones_h = jnp.ones((CH, HALF), jnp.float32)

  degA, degB = _sc_deg(dst, ones_h, zrow)
  aggA, aggB = _sc_segsum(xA, xB, src, dst, zrow, HALF)
  y2A, y2B = _tc_stage1(x, aggA, aggB, degA, degB,
                        W1, b1.reshape(1, HID), W2, b2.reshape(1, OUTF))
  agg2A, agg2B = _sc_segsum(y2A, y2B, src, dst, zrow, HALF)
  h = _tc_stage2(y2A, y2B, agg2A, agg2B, degA, degB)

  bidx_rep = jnp.repeat(sample_batch, KK)
  gidx = jnp.concatenate([bidx_rep, shuffled_index.reshape(-1)])
  rows = _sc_gather(h, gidx)
  ha_rows = rows[:BB * KK]
  hb_rows = rows[BB * KK:]
  dec = _tc_decoder(ha_rows, hb_rows, Wd, bd.reshape(1, DECF)).reshape(BB, KK)
  return (h, dec)


# speedup vs baseline: 4.6011x; 1.0653x over previous
import functools

import jax
import jax.numpy as jnp
from jax import lax
from jax.experimental import pallas as pl
from jax.experimental.pallas import tpu as pltpu
from jax.experimental.pallas import tpu_sc as plsc

N = 10000
E = 160000
D_IN = 256
HID = 512
OUTF = 256
DECF = 256
BB = 1024
KK = 16

NC = 2
NS = 16
NW = NC * NS
HALF = D_IN // 2
CH = 128
NCHUNK = E // CH
QPT = -(-NCHUNK // NS)
G = 3
RW = 200
NRCH = N // RW
RCPT = -(-NRCH // NS)

_SC_MESH = dict(core_axis_name="c", subcore_axis_name="s",
                num_cores=NC, num_subcores=NS)


def _row_chunks(s, fn):
  def body(j, carry):
    cid = s + NS * j

    @pl.when(cid < NRCH)
    def _():
      fn(cid * RW)
    return carry

  lax.fori_loop(0, RCPT, body, 0)


def _seg_pipe(s, tab_h, src_h, dst_h, sidx, didx, rows, semi, semj, semg,
              sems, acc_s):
  lo = jnp.minimum(s * QPT, NCHUNK)
  hi = jnp.minimum(lo + QPT, NCHUNK)
  ngroups = (hi - lo) // G

  def drain_scatter(b):
    pltpu.make_async_copy(tab_h.at[pl.ds(0, CH)], rows[b], sems.at[b]).wait()

  def group(g, carry):
    cid0 = lo + g * G
    di = []
    dj = []
    for b in range(G):
      off = (cid0 + b) * CH

      @pl.when(g > 0)
      def _(b=b):
        drain_scatter(b)
      di.append(pltpu.async_copy(src_h.at[pl.ds(off, CH)], sidx[b],
                                 semi.at[b]))
      dj.append(pltpu.async_copy(dst_h.at[pl.ds(off, CH)], didx[b],
                                 semj.at[b]))
    dg = []
    for b in range(G):
      di[b].wait()
      dg.append(pltpu.async_copy(tab_h.at[sidx[b]], rows[b], semg.at[b]))
    for b in range(G):
      dg[b].wait()
      dj[b].wait()
      pltpu.async_copy(rows[b], acc_s.at[didx[b]], sems.at[b], add=True)
    return carry

  lax.fori_loop(0, ngroups, group, 0)
  for b in range(G):
    @pl.when(ngroups > 0)
    def _(b=b):
      drain_scatter(b)

  def tail(tt, carry):
    cid = lo + ngroups * G + tt
    off = cid * CH
    pltpu.sync_copy(src_h.at[pl.ds(off, CH)], sidx[0])
    pltpu.sync_copy(dst_h.at[pl.ds(off, CH)], didx[0])
    pltpu.async_copy(tab_h.at[sidx[0]], rows[0], semg.at[0]).wait()
    pltpu.sync_copy(rows[0], acc_s.at[didx[0]], add=True)
    return carry

  lax.fori_loop(0, hi - lo - ngroups * G, tail, 0)


def _sc_segsum(xA, xB, src, dst, zrow, w):
  @functools.partial(
      pl.kernel,
      out_type=(jax.ShapeDtypeStruct((N, w), jnp.float32),
                jax.ShapeDtypeStruct((N, w), jnp.float32)),
      mesh=plsc.VectorSubcoreMesh(**_SC_MESH),
      scratch_types=(
          [pltpu.VMEM((CH,), jnp.int32)] * G +
          [pltpu.VMEM((CH,), jnp.int32)] * G +
          [pltpu.VMEM((CH, w), jnp.float32)] * G +
          [pltpu.VMEM_SHARED((N, w), jnp.float32),
           pltpu.SemaphoreType.DMA((G,)),
           pltpu.SemaphoreType.DMA((G,)),
           pltpu.SemaphoreType.DMA((G,)),
           pltpu.SemaphoreType.DMA((G,))]),
  )
  def k(xA_h, xB_h, src_h, dst_h, zrow_h, aggA_h, aggB_h, *refs):
    sidx = refs[:G]
    didx = refs[G:2 * G]
    rows = refs[2 * G:3 * G]
    acc_s, semi, semj, semg, sems = refs[3 * G:]
    c = lax.axis_index("c")
    s = lax.axis_index("s")
    _row_chunks(s, lambda off: pltpu.sync_copy(zrow_h, acc_s.at[pl.ds(off, RW)]))
    plsc.subcore_barrier()

    @pl.when(c == 0)
    def _():
      _seg_pipe(s, xA_h, src_h, dst_h, sidx, didx, rows, semi, semj, semg,
                sems, acc_s)

    @pl.when(c == 1)
    def _():
      _seg_pipe(s, xB_h, src_h, dst_h, sidx, didx, rows, semi, semj, semg,
                sems, acc_s)

    plsc.subcore_barrier()

    def wb(off):
      @pl.when(c == 0)
      def _():
        pltpu.sync_copy(acc_s.at[pl.ds(off, RW)], aggA_h.at[pl.ds(off, RW)])

      @pl.when(c == 1)
      def _():
        pltpu.sync_copy(acc_s.at[pl.ds(off, RW)], aggB_h.at[pl.ds(off, RW)])

    _row_chunks(s, wb)

  return k(xA, xB, src, dst, zrow)


DCPT = -(-NCHUNK // NW)


def _sc_deg(dst, ones_h, zrow):
  @functools.partial(
      pl.kernel,
      out_type=(jax.ShapeDtypeStruct((N, HALF), jnp.float32),
                jax.ShapeDtypeStruct((N, HALF), jnp.float32)),
      mesh=plsc.VectorSubcoreMesh(**_SC_MESH),
      scratch_types=[
          pltpu.VMEM((CH,), jnp.int32),
          pltpu.VMEM((CH, HALF), jnp.float32),
          pltpu.VMEM_SHARED((N, HALF), jnp.float32),
      ],
  )
  def k(dst_h, ones_hh, zrow_h, degA_h, degB_h, dsti_v, ones_v, acc_s):
    c = lax.axis_index("c")
    s = lax.axis_index("s")
    wid = s * NC + c
    _row_chunks(s, lambda off: pltpu.sync_copy(zrow_h, acc_s.at[pl.ds(off, RW)]))
    pltpu.sync_copy(ones_hh, ones_v)
    plsc.subcore_barrier()

    def body(j, carry):
      cid = wid + NW * j

      @pl.when(cid < NCHUNK)
      def _():
        off = cid * CH
        pltpu.sync_copy(dst_h.at[pl.ds(off, CH)], dsti_v)
        pltpu.sync_copy(ones_v, acc_s.at[dsti_v], add=True)
      return carry

    lax.fori_loop(0, DCPT, body, 0)
    plsc.subcore_barrier()

    def wb(off):
      @pl.when(c == 0)
      def _():
        pltpu.sync_copy(acc_s.at[pl.ds(off, RW)], degA_h.at[pl.ds(off, RW)])

      @pl.when(c == 1)
      def _():
        pltpu.sync_copy(acc_s.at[pl.ds(off, RW)], degB_h.at[pl.ds(off, RW)])

    _row_chunks(s, wb)

  return k(dst, ones_h, zrow)


NG = 32768
GPT = NG // NW
GCH = 128


def _sc_gather(h, idx):
  @functools.partial(
      pl.kernel,
      out_type=jax.ShapeDtypeStruct((NG, OUTF), jnp.float32),
      mesh=plsc.VectorSubcoreMesh(**_SC_MESH),
      scratch_types=[
          pltpu.VMEM((GCH,), jnp.int32),
          pltpu.VMEM((GCH, OUTF), jnp.float32),
          pltpu.SemaphoreType.DMA,
      ],
  )
  def k(h_h, idx_h, out_h, idx_v, rows_v, sem):
    c = lax.axis_index("c")
    s = lax.axis_index("s")
    wid = s * NC + c

    def body(j, carry):
      off = wid * GPT + j * GCH
      pltpu.sync_copy(idx_h.at[pl.ds(off, GCH)], idx_v)
      pltpu.async_copy(h_h.at[idx_v], rows_v, sem).wait()
      pltpu.sync_copy(rows_v, out_h.at[pl.ds(off, GCH)])
      return carry

    lax.fori_loop(0, GPT // GCH, body, 0)

  return k(h, idx)


RB = 400


def _tc_stage1(x, aggA, aggB, degA, degB, W1, b1, W2, b2):
  def body(x_r, aA_r, aB_r, dA_r, dB_r, W1_r, b1_r, W2_r, b2_r, oA_r, oB_r):
    agg = jnp.concatenate([aA_r[...], aB_r[...]], axis=1)
    rec = 1.0 / (dA_r[...][:, 0:1] + dB_r[...][:, 0:1] + 1.0)
    z1 = (agg + x_r[...]) * rec
    h1 = jnp.dot(z1, W1_r[...], preferred_element_type=jnp.float32) + b1_r[...]
    h1 = jnp.maximum(h1, 0.0)
    y2 = jnp.dot(h1, W2_r[...], preferred_element_type=jnp.float32) + b2_r[...]
    oA_r[...] = y2[:, :HALF]
    oB_r[...] = y2[:, HALF:]

  return pl.pallas_call(
      body,
      grid=(N // RB,),
      in_specs=[
          pl.BlockSpec((RB, D_IN), lambda i: (i, 0)),
          pl.BlockSpec((RB, HALF), lambda i: (i, 0)),
          pl.BlockSpec((RB, HALF), lambda i: (i, 0)),
          pl.BlockSpec((RB, HALF), lambda i: (i, 0)),
          pl.BlockSpec((RB, HALF), lambda i: (i, 0)),
          pl.BlockSpec((D_IN, HID), lambda i: (0, 0)),
          pl.BlockSpec((1, HID), lambda i: (0, 0)),
          pl.BlockSpec((HID, OUTF), lambda i: (0, 0)),
          pl.BlockSpec((1, OUTF), lambda i: (0, 0)),
      ],
      out_specs=[
          pl.BlockSpec((RB, HALF), lambda i: (i, 0)),
          pl.BlockSpec((RB, HALF), lambda i: (i, 0)),
      ],
      out_shape=[
          jax.ShapeDtypeStruct((N, HALF), jnp.float32),
          jax.ShapeDtypeStruct((N, HALF), jnp.float32),
      ],
  )(x, aggA, aggB, degA, degB, W1, b1, W2, b2)


def _tc_stage2(y2A, y2B, aggA, aggB, degA, degB):
  def body(yA_r, yB_r, aA_r, aB_r, dA_r, dB_r, h_r):
    rec = 1.0 / (dA_r[...][:, 0:1] + dB_r[...][:, 0:1] + 1.0)
    left = (aA_r[...] + yA_r[...]) * rec
    right = (aB_r[...] + yB_r[...]) * rec
    h_r[...] = jnp.concatenate([left, right], axis=1)

  return pl.pallas_call(
      body,
      grid=(N // RB,),
      in_specs=[pl.BlockSpec((RB, HALF), lambda i: (i, 0))] * 6,
      out_specs=pl.BlockSpec((RB, OUTF), lambda i: (i, 0)),
      out_shape=jax.ShapeDtypeStruct((N, OUTF), jnp.float32),
  )(y2A, y2B, aggA, aggB, degA, degB)


DB = 2048


def _tc_decoder(ha_rows, hb_rows, Wd, bd):
  def body(a_r, b_r, Wd_r, bd_r, o_r):
    ha = jnp.dot(a_r[...], Wd_r[...], preferred_element_type=jnp.float32) + bd_r[...]
    hb = jnp.dot(b_r[...], Wd_r[...], preferred_element_type=jnp.float32) + bd_r[...]
    num = jnp.sum(ha * hb, axis=1, keepdims=True)
    na = jnp.sqrt(jnp.sum(ha * ha, axis=1, keepdims=True))
    nb = jnp.sqrt(jnp.sum(hb * hb, axis=1, keepdims=True))
    o_r[...] = num / jnp.maximum(na * nb, 1e-8)

  return pl.pallas_call(
      body,
      grid=(BB * KK // DB,),
      in_specs=[
          pl.BlockSpec((DB, OUTF), lambda i: (i, 0)),
          pl.BlockSpec((DB, OUTF), lambda i: (i, 0)),
          pl.BlockSpec((OUTF, DECF), lambda i: (0, 0)),
          pl.BlockSpec((1, DECF), lambda i: (0, 0)),
      ],
      out_specs=pl.BlockSpec((DB, 1), lambda i: (i, 0)),
      out_shape=jax.ShapeDtypeStruct((BB * KK, 1), jnp.float32),
  )(ha_rows, hb_rows, Wd, bd)


def kernel(x, edge_index, shuffled_index, sample_batch, W1, b1, W2, b2, Wd, bd):
  src = edge_index[0]
  dst = edge_index[1]
  xA = x[:, :HALF]
  xB = x[:, HALF:]
  zrow = jnp.zeros((RW, HALF), jnp.float32)
  ones_h = jnp.ones((CH, HALF), jnp.float32)

  degA, degB = _sc_deg(dst, ones_h, zrow)
  aggA, aggB = _sc_segsum(xA, xB, src, dst, zrow, HALF)
  y2A, y2B = _tc_stage1(x, aggA, aggB, degA, degB,
                        W1, b1.reshape(1, HID), W2, b2.reshape(1, OUTF))
  agg2A, agg2B = _sc_segsum(y2A, y2B, src, dst, zrow, HALF)
  h = _tc_stage2(y2A, y2B, agg2A, agg2B, degA, degB)

  bidx_rep = jnp.repeat(sample_batch, KK)
  gidx = jnp.concatenate([bidx_rep, shuffled_index.reshape(-1)])
  rows = _sc_gather(h, gidx)
  ha_rows = rows[:BB * KK]
  hb_rows = rows[BB * KK:]
  dec = _tc_decoder(ha_rows, hb_rows, Wd, bd.reshape(1, DECF)).reshape(BB, KK)
  return (h, dec)


# speedup vs baseline: 5.2265x; 1.1359x over previous
import functools

import jax
import jax.numpy as jnp
from jax import lax
from jax.experimental import pallas as pl
from jax.experimental.pallas import tpu as pltpu
from jax.experimental.pallas import tpu_sc as plsc

N = 10000
E = 160000
D_IN = 256
HID = 512
OUTF = 256
DECF = 256
BB = 1024
KK = 16

NC = 2
NS = 16
NW = NC * NS
HALF = D_IN // 2
CH = 128
NCHUNK = E // CH
QPT = -(-NCHUNK // NS)
G = 3
RW = 200
NRCH = N // RW
RCPT = -(-NRCH // NS)

_SC_MESH = dict(core_axis_name="c", subcore_axis_name="s",
                num_cores=NC, num_subcores=NS)


def _row_chunks(s, fn):
  def body(j, carry):
    cid = s + NS * j

    @pl.when(cid < NRCH)
    def _():
      fn(cid * RW)
    return carry

  lax.fori_loop(0, RCPT, body, 0)


def _seg_pipe(s, tab_h, src_h, dst_h, sidx, didx, rows, semi, semj, semg,
              sems, acc_s):
  lo = jnp.minimum(s * QPT, NCHUNK)
  hi = jnp.minimum(lo + QPT, NCHUNK)
  ngroups = (hi - lo) // G

  def drain_scatter(b):
    pltpu.make_async_copy(tab_h.at[pl.ds(0, CH)], rows[b], sems.at[b]).wait()

  def group(g, carry):
    cid0 = lo + g * G
    di = []
    dj = []
    for b in range(G):
      off = (cid0 + b) * CH

      @pl.when(g > 0)
      def _(b=b):
        drain_scatter(b)
      di.append(pltpu.async_copy(src_h.at[pl.ds(off, CH)], sidx[b],
                                 semi.at[b]))
      dj.append(pltpu.async_copy(dst_h.at[pl.ds(off, CH)], didx[b],
                                 semj.at[b]))
    dg = []
    for b in range(G):
      di[b].wait()
      dg.append(pltpu.async_copy(tab_h.at[sidx[b]], rows[b], semg.at[b]))
    for b in range(G):
      dg[b].wait()
      dj[b].wait()
      pltpu.async_copy(rows[b], acc_s.at[didx[b]], sems.at[b], add=True)
    return carry

  lax.fori_loop(0, ngroups, group, 0)
  for b in range(G):
    @pl.when(ngroups > 0)
    def _(b=b):
      drain_scatter(b)

  def tail(tt, carry):
    cid = lo + ngroups * G + tt
    off = cid * CH
    pltpu.sync_copy(src_h.at[pl.ds(off, CH)], sidx[0])
    pltpu.sync_copy(dst_h.at[pl.ds(off, CH)], didx[0])
    pltpu.async_copy(tab_h.at[sidx[0]], rows[0], semg.at[0]).wait()
    pltpu.sync_copy(rows[0], acc_s.at[didx[0]], add=True)
    return carry

  lax.fori_loop(0, hi - lo - ngroups * G, tail, 0)


def _sc_segsum(xA, xB, src, dst, zrow, w):
  @functools.partial(
      pl.kernel,
      out_type=(jax.ShapeDtypeStruct((N, w), jnp.float32),
                jax.ShapeDtypeStruct((N, w), jnp.float32)),
      mesh=plsc.VectorSubcoreMesh(**_SC_MESH),
      scratch_types=(
          [pltpu.VMEM((CH,), jnp.int32)] * G +
          [pltpu.VMEM((CH,), jnp.int32)] * G +
          [pltpu.VMEM((CH, w), jnp.float32)] * G +
          [pltpu.VMEM_SHARED((N, w), jnp.float32),
           pltpu.SemaphoreType.DMA((G,)),
           pltpu.SemaphoreType.DMA((G,)),
           pltpu.SemaphoreType.DMA((G,)),
           pltpu.SemaphoreType.DMA((G,))]),
  )
  def k(xA_h, xB_h, src_h, dst_h, zrow_h, aggA_h, aggB_h, *refs):
    sidx = refs[:G]
    didx = refs[G:2 * G]
    rows = refs[2 * G:3 * G]
    acc_s, semi, semj, semg, sems = refs[3 * G:]
    c = lax.axis_index("c")
    s = lax.axis_index("s")
    _row_chunks(s, lambda off: pltpu.sync_copy(zrow_h, acc_s.at[pl.ds(off, RW)]))
    plsc.subcore_barrier()

    @pl.when(c == 0)
    def _():
      _seg_pipe(s, xA_h, src_h, dst_h, sidx, didx, rows, semi, semj, semg,
                sems, acc_s)

    @pl.when(c == 1)
    def _():
      _seg_pipe(s, xB_h, src_h, dst_h, sidx, didx, rows, semi, semj, semg,
                sems, acc_s)

    plsc.subcore_barrier()

    def wb(off):
      @pl.when(c == 0)
      def _():
        pltpu.sync_copy(acc_s.at[pl.ds(off, RW)], aggA_h.at[pl.ds(off, RW)])

      @pl.when(c == 1)
      def _():
        pltpu.sync_copy(acc_s.at[pl.ds(off, RW)], aggB_h.at[pl.ds(off, RW)])

    _row_chunks(s, wb)

  return k(xA, xB, src, dst, zrow)


DQ = -(-NCHUNK // NW)
DG = 4


def _sc_deg(dst, ones_h, zrow):
  @functools.partial(
      pl.kernel,
      out_type=(jax.ShapeDtypeStruct((N, HALF), jnp.float32),
                jax.ShapeDtypeStruct((N, HALF), jnp.float32)),
      mesh=plsc.VectorSubcoreMesh(**_SC_MESH),
      scratch_types=(
          [pltpu.VMEM((CH,), jnp.int32)] * DG +
          [pltpu.VMEM((CH, HALF), jnp.float32),
           pltpu.VMEM_SHARED((N, HALF), jnp.float32),
           pltpu.SemaphoreType.DMA((DG,)),
           pltpu.SemaphoreType.DMA((DG,))]),
  )
  def k(dst_h, ones_hh, zrow_h, degA_h, degB_h, *refs):
    didx = refs[:DG]
    ones_v, acc_s, semd, semsc = refs[DG:]
    c = lax.axis_index("c")
    s = lax.axis_index("s")
    wid = s * NC + c
    _row_chunks(s, lambda off: pltpu.sync_copy(zrow_h, acc_s.at[pl.ds(off, RW)]))
    pltpu.sync_copy(ones_hh, ones_v)
    plsc.subcore_barrier()

    lo = jnp.minimum(wid * DQ, NCHUNK)
    hi = jnp.minimum(lo + DQ, NCHUNK)
    ngroups = (hi - lo) // DG

    def drain_scatter(b):
      pltpu.make_async_copy(ones_hh, ones_v, semsc.at[b]).wait()

    def group(g, carry):
      cid0 = lo + g * DG
      dd = []
      for b in range(DG):
        @pl.when(g > 0)
        def _(b=b):
          drain_scatter(b)
        dd.append(pltpu.async_copy(dst_h.at[pl.ds((cid0 + b) * CH, CH)],
                                   didx[b], semd.at[b]))
      for b in range(DG):
        dd[b].wait()
        pltpu.async_copy(ones_v, acc_s.at[didx[b]], semsc.at[b], add=True)
      return carry

    lax.fori_loop(0, ngroups, group, 0)
    for b in range(DG):
      @pl.when(ngroups > 0)
      def _(b=b):
        drain_scatter(b)

    def tail(tt, carry):
      cid = lo + ngroups * DG + tt
      pltpu.sync_copy(dst_h.at[pl.ds(cid * CH, CH)], didx[0])
      pltpu.sync_copy(ones_v, acc_s.at[didx[0]], add=True)
      return carry

    lax.fori_loop(0, hi - lo - ngroups * DG, tail, 0)
    plsc.subcore_barrier()

    def wb(off):
      @pl.when(c == 0)
      def _():
        pltpu.sync_copy(acc_s.at[pl.ds(off, RW)], degA_h.at[pl.ds(off, RW)])

      @pl.when(c == 1)
      def _():
        pltpu.sync_copy(acc_s.at[pl.ds(off, RW)], degB_h.at[pl.ds(off, RW)])

    _row_chunks(s, wb)

  return k(dst, ones_h, zrow)


HBCH = 4


def _sc_gather(h, bidx, sidxt):
  @functools.partial(
      pl.kernel,
      out_type=(jax.ShapeDtypeStruct((BB, OUTF), jnp.float32),
                jax.ShapeDtypeStruct((BB * KK, OUTF), jnp.float32)),
      mesh=plsc.VectorSubcoreMesh(**_SC_MESH),
      scratch_types=(
          [pltpu.VMEM((CH,), jnp.int32)] * (HBCH + 1) +
          [pltpu.VMEM((CH, OUTF), jnp.float32)] * 3 +
          [pltpu.SemaphoreType.DMA((HBCH + 1,)),
           pltpu.SemaphoreType.DMA((HBCH + 1,)),
           pltpu.SemaphoreType.DMA((HBCH + 1,))]),
  )
  def k(h_h, bidx_h, sidxt_h, ha_h, hb_h, *refs):
    idxb = refs[:HBCH + 1]
    rows = refs[HBCH + 1:HBCH + 4]
    semi, semg, semw = refs[HBCH + 4:]
    c = lax.axis_index("c")
    s = lax.axis_index("s")
    wid = s * NC + c

    di = []
    for q in range(HBCH):
      di.append(pltpu.async_copy(
          sidxt_h.at[pl.ds(wid * (HBCH * CH) + q * CH, CH)], idxb[q],
          semi.at[q]))
    dg = {}
    dw = {}
    waited = set()
    for q in range(HBCH + 1):
      if q < HBCH:
        b = q % 3
        if q >= 3:
          dw[q - 3].wait()
          waited.add(q - 3)
        di[q].wait()
        dg[q] = pltpu.async_copy(h_h.at[idxb[q]], rows[b], semg.at[q])
      if 1 <= q:
        p = q - 1
        dg[p].wait()
        dw[p] = pltpu.async_copy(
            rows[p % 3], hb_h.at[pl.ds(wid * (HBCH * CH) + p * CH, CH)],
            semw.at[p])
    for q in range(HBCH):
      if q not in waited:
        dw[q].wait()

    @pl.when(wid < BB // CH)
    def _():
      pltpu.sync_copy(bidx_h.at[pl.ds(wid * CH, CH)], idxb[HBCH])
      pltpu.async_copy(h_h.at[idxb[HBCH]], rows[0], semg.at[HBCH]).wait()
      pltpu.sync_copy(rows[0], ha_h.at[pl.ds(wid * CH, CH)])

  return k(h, bidx, sidxt)


RB = 400


def _tc_stage1(x, aggA, aggB, degA, degB, W1, b1, W2, b2):
  def body(x_r, aA_r, aB_r, dA_r, dB_r, W1_r, b1_r, W2_r, b2_r, oA_r, oB_r):
    agg = jnp.concatenate([aA_r[...], aB_r[...]], axis=1)
    rec = 1.0 / (dA_r[...][:, 0:1] + dB_r[...][:, 0:1] + 1.0)
    z1 = (agg + x_r[...]) * rec
    h1 = jnp.dot(z1, W1_r[...], preferred_element_type=jnp.float32) + b1_r[...]
    h1 = jnp.maximum(h1, 0.0)
    y2 = jnp.dot(h1, W2_r[...], preferred_element_type=jnp.float32) + b2_r[...]
    oA_r[...] = y2[:, :HALF]
    oB_r[...] = y2[:, HALF:]

  return pl.pallas_call(
      body,
      grid=(N // RB,),
      in_specs=[
          pl.BlockSpec((RB, D_IN), lambda i: (i, 0)),
          pl.BlockSpec((RB, HALF), lambda i: (i, 0)),
          pl.BlockSpec((RB, HALF), lambda i: (i, 0)),
          pl.BlockSpec((RB, HALF), lambda i: (i, 0)),
          pl.BlockSpec((RB, HALF), lambda i: (i, 0)),
          pl.BlockSpec((D_IN, HID), lambda i: (0, 0)),
          pl.BlockSpec((1, HID), lambda i: (0, 0)),
          pl.BlockSpec((HID, OUTF), lambda i: (0, 0)),
          pl.BlockSpec((1, OUTF), lambda i: (0, 0)),
      ],
      out_specs=[
          pl.BlockSpec((RB, HALF), lambda i: (i, 0)),
          pl.BlockSpec((RB, HALF), lambda i: (i, 0)),
      ],
      out_shape=[
          jax.ShapeDtypeStruct((N, HALF), jnp.float32),
          jax.ShapeDtypeStruct((N, HALF), jnp.float32),
      ],
  )(x, aggA, aggB, degA, degB, W1, b1, W2, b2)


def _tc_stage2(y2A, y2B, aggA, aggB, degA, degB):
  def body(yA_r, yB_r, aA_r, aB_r, dA_r, dB_r, h_r):
    rec = 1.0 / (dA_r[...][:, 0:1] + dB_r[...][:, 0:1] + 1.0)
    left = (aA_r[...] + yA_r[...]) * rec
    right = (aB_r[...] + yB_r[...]) * rec
    h_r[...] = jnp.concatenate([left, right], axis=1)

  return pl.pallas_call(
      body,
      grid=(N // RB,),
      in_specs=[pl.BlockSpec((RB, HALF), lambda i: (i, 0))] * 6,
      out_specs=pl.BlockSpec((RB, OUTF), lambda i: (i, 0)),
      out_shape=jax.ShapeDtypeStruct((N, OUTF), jnp.float32),
  )(y2A, y2B, aggA, aggB, degA, degB)


DBB = 512


def _tc_ha(ha_rows, Wd, bd):
  def body(a_r, Wd_r, bd_r, o_r, n_r):
    had = jnp.dot(a_r[...], Wd_r[...],
                  preferred_element_type=jnp.float32) + bd_r[...]
    o_r[...] = had
    n_r[...] = jnp.sum(had * had, axis=1, keepdims=True)

  return pl.pallas_call(
      body,
      grid=(BB // DBB,),
      in_specs=[
          pl.BlockSpec((DBB, OUTF), lambda i: (i, 0)),
          pl.BlockSpec((OUTF, DECF), lambda i: (0, 0)),
          pl.BlockSpec((1, DECF), lambda i: (0, 0)),
      ],
      out_specs=[
          pl.BlockSpec((DBB, DECF), lambda i: (i, 0)),
          pl.BlockSpec((DBB, 1), lambda i: (i, 0)),
      ],
      out_shape=[
          jax.ShapeDtypeStruct((BB, DECF), jnp.float32),
          jax.ShapeDtypeStruct((BB, 1), jnp.float32),
      ],
  )(ha_rows, Wd, bd)


def _tc_decoder(ha_dec, na2, hb_rows, Wd, bd):
  def body(a_r, n_r, b_r, Wd_r, bd_r, o_r):
    hb = jnp.dot(b_r[...], Wd_r[...],
                 preferred_element_type=jnp.float32) + bd_r[...]
    num = jnp.sum(a_r[...] * hb, axis=1, keepdims=True)
    nb2 = jnp.sum(hb * hb, axis=1, keepdims=True)
    o_r[...] = num / jnp.maximum(jnp.sqrt(n_r[...] * nb2), 1e-8)

  nb = BB // DBB
  return pl.pallas_call(
      body,
      grid=(KK, nb),
      in_specs=[
          pl.BlockSpec((DBB, DECF), lambda k, i: (i, 0)),
          pl.BlockSpec((DBB, 1), lambda k, i: (i, 0)),
          pl.BlockSpec((DBB, OUTF), lambda k, i: (k * (BB // DBB) + i, 0)),
          pl.BlockSpec((OUTF, DECF), lambda k, i: (0, 0)),
          pl.BlockSpec((1, DECF), lambda k, i: (0, 0)),
      ],
      out_specs=pl.BlockSpec((DBB, 1), lambda k, i: (k * (BB // DBB) + i, 0)),
      out_shape=jax.ShapeDtypeStruct((BB * KK, 1), jnp.float32),
  )(ha_dec, na2, hb_rows, Wd, bd)


def kernel(x, edge_index, shuffled_index, sample_batch, W1, b1, W2, b2, Wd, bd):
  src = edge_index[0]
  dst = edge_index[1]
  xA = x[:, :HALF]
  xB = x[:, HALF:]
  zrow = jnp.zeros((RW, HALF), jnp.float32)
  ones_h = jnp.ones((CH, HALF), jnp.float32)

  degA, degB = _sc_deg(dst, ones_h, zrow)
  aggA, aggB = _sc_segsum(xA, xB, src, dst, zrow, HALF)
  y2A, y2B = _tc_stage1(x, aggA, aggB, degA, degB,
                        W1, b1.reshape(1, HID), W2, b2.reshape(1, OUTF))
  agg2A, agg2B = _sc_segsum(y2A, y2B, src, dst, zrow, HALF)
  h = _tc_stage2(y2A, y2B, agg2A, agg2B, degA, degB)

  sidxt = shuffled_index.T.reshape(-1)
  ha_rows, hb_rows = _sc_gather(h, sample_batch, sidxt)
  ha_dec, na2 = _tc_ha(ha_rows, Wd, bd.reshape(1, DECF))
  dec_t = _tc_decoder(ha_dec, na2, hb_rows, Wd, bd.reshape(1, DECF))
  dec = dec_t.reshape(KK, BB).T
  return (h, dec)


# speedup vs baseline: 5.2561x; 1.0057x over previous
import functools

import jax
import jax.numpy as jnp
from jax import lax
from jax.experimental import pallas as pl
from jax.experimental.pallas import tpu as pltpu
from jax.experimental.pallas import tpu_sc as plsc

N = 10000
E = 160000
D_IN = 256
HID = 512
OUTF = 256
DECF = 256
BB = 1024
KK = 16

NC = 2
NS = 16
NW = NC * NS
HALF = D_IN // 2
CH = 128
NCHUNK = E // CH
SCH = 64
NCHUNK_S = E // SCH
QPT = -(-NCHUNK_S // NS)
G = 6
RW = 200
NRCH = N // RW
RCPT = -(-NRCH // NS)

_SC_MESH = dict(core_axis_name="c", subcore_axis_name="s",
                num_cores=NC, num_subcores=NS)


def _row_chunks(s, fn):
  def body(j, carry):
    cid = s + NS * j

    @pl.when(cid < NRCH)
    def _():
      fn(cid * RW)
    return carry

  lax.fori_loop(0, RCPT, body, 0)


def _seg_pipe(s, tab_h, src_h, dst_h, sidx, didx, rows, semi, semj, semg,
              sems, acc_s):
  lo = jnp.minimum(s * QPT, NCHUNK_S)
  hi = jnp.minimum(lo + QPT, NCHUNK_S)
  ngroups = (hi - lo) // G

  def drain_scatter(b):
    pltpu.make_async_copy(tab_h.at[pl.ds(0, SCH)], rows[b], sems.at[b]).wait()

  def group(g, carry):
    cid0 = lo + g * G
    di = []
    dj = []
    for b in range(G):
      off = (cid0 + b) * SCH

      @pl.when(g > 0)
      def _(b=b):
        drain_scatter(b)
      di.append(pltpu.async_copy(src_h.at[pl.ds(off, SCH)], sidx[b],
                                 semi.at[b]))
      dj.append(pltpu.async_copy(dst_h.at[pl.ds(off, SCH)], didx[b],
                                 semj.at[b]))
    dg = []
    for b in range(G):
      di[b].wait()
      dg.append(pltpu.async_copy(tab_h.at[sidx[b]], rows[b], semg.at[b]))
    for b in range(G):
      dg[b].wait()
      dj[b].wait()
      pltpu.async_copy(rows[b], acc_s.at[didx[b]], sems.at[b], add=True)
    return carry

  lax.fori_loop(0, ngroups, group, 0)
  for b in range(G):
    @pl.when(ngroups > 0)
    def _(b=b):
      drain_scatter(b)

  def tail(tt, carry):
    cid = lo + ngroups * G + tt
    off = cid * SCH
    pltpu.sync_copy(src_h.at[pl.ds(off, SCH)], sidx[0])
    pltpu.sync_copy(dst_h.at[pl.ds(off, SCH)], didx[0])
    pltpu.async_copy(tab_h.at[sidx[0]], rows[0], semg.at[0]).wait()
    pltpu.sync_copy(rows[0], acc_s.at[didx[0]], add=True)
    return carry

  lax.fori_loop(0, hi - lo - ngroups * G, tail, 0)


def _sc_segsum(xA, xB, src, dst, zrow, w):
  @functools.partial(
      pl.kernel,
      out_type=(jax.ShapeDtypeStruct((N, w), jnp.float32),
                jax.ShapeDtypeStruct((N, w), jnp.float32)),
      mesh=plsc.VectorSubcoreMesh(**_SC_MESH),
      scratch_types=(
          [pltpu.VMEM((SCH,), jnp.int32)] * G +
          [pltpu.VMEM((SCH,), jnp.int32)] * G +
          [pltpu.VMEM((SCH, w), jnp.float32)] * G +
          [pltpu.VMEM_SHARED((N, w), jnp.float32),
           pltpu.SemaphoreType.DMA((G,)),
           pltpu.SemaphoreType.DMA((G,)),
           pltpu.SemaphoreType.DMA((G,)),
           pltpu.SemaphoreType.DMA((G,))]),
  )
  def k(xA_h, xB_h, src_h, dst_h, zrow_h, aggA_h, aggB_h, *refs):
    sidx = refs[:G]
    didx = refs[G:2 * G]
    rows = refs[2 * G:3 * G]
    acc_s, semi, semj, semg, sems = refs[3 * G:]
    c = lax.axis_index("c")
    s = lax.axis_index("s")
    _row_chunks(s, lambda off: pltpu.sync_copy(zrow_h, acc_s.at[pl.ds(off, RW)]))
    plsc.subcore_barrier()

    @pl.when(c == 0)
    def _():
      _seg_pipe(s, xA_h, src_h, dst_h, sidx, didx, rows, semi, semj, semg,
                sems, acc_s)

    @pl.when(c == 1)
    def _():
      _seg_pipe(s, xB_h, src_h, dst_h, sidx, didx, rows, semi, semj, semg,
                sems, acc_s)

    plsc.subcore_barrier()

    def wb(off):
      @pl.when(c == 0)
      def _():
        pltpu.sync_copy(acc_s.at[pl.ds(off, RW)], aggA_h.at[pl.ds(off, RW)])

      @pl.when(c == 1)
      def _():
        pltpu.sync_copy(acc_s.at[pl.ds(off, RW)], aggB_h.at[pl.ds(off, RW)])

    _row_chunks(s, wb)

  return k(xA, xB, src, dst, zrow)


DQ = -(-NCHUNK // NW)
DG = 4


def _sc_deg(dst, ones_h, zrow):
  @functools.partial(
      pl.kernel,
      out_type=(jax.ShapeDtypeStruct((N, HALF), jnp.float32),
                jax.ShapeDtypeStruct((N, HALF), jnp.float32)),
      mesh=plsc.VectorSubcoreMesh(**_SC_MESH),
      scratch_types=(
          [pltpu.VMEM((CH,), jnp.int32)] * DG +
          [pltpu.VMEM((CH, HALF), jnp.float32),
           pltpu.VMEM_SHARED((N, HALF), jnp.float32),
           pltpu.SemaphoreType.DMA((DG,)),
           pltpu.SemaphoreType.DMA((DG,))]),
  )
  def k(dst_h, ones_hh, zrow_h, degA_h, degB_h, *refs):
    didx = refs[:DG]
    ones_v, acc_s, semd, semsc = refs[DG:]
    c = lax.axis_index("c")
    s = lax.axis_index("s")
    wid = s * NC + c
    _row_chunks(s, lambda off: pltpu.sync_copy(zrow_h, acc_s.at[pl.ds(off, RW)]))
    pltpu.sync_copy(ones_hh, ones_v)
    plsc.subcore_barrier()

    lo = jnp.minimum(wid * DQ, NCHUNK)
    hi = jnp.minimum(lo + DQ, NCHUNK)
    ngroups = (hi - lo) // DG

    def drain_scatter(b):
      pltpu.make_async_copy(ones_hh, ones_v, semsc.at[b]).wait()

    def group(g, carry):
      cid0 = lo + g * DG
      dd = []
      for b in range(DG):
        @pl.when(g > 0)
        def _(b=b):
          drain_scatter(b)
        dd.append(pltpu.async_copy(dst_h.at[pl.ds((cid0 + b) * CH, CH)],
                                   didx[b], semd.at[b]))
      for b in range(DG):
        dd[b].wait()
        pltpu.async_copy(ones_v, acc_s.at[didx[b]], semsc.at[b], add=True)
      return carry

    lax.fori_loop(0, ngroups, group, 0)
    for b in range(DG):
      @pl.when(ngroups > 0)
      def _(b=b):
        drain_scatter(b)

    def tail(tt, carry):
      cid = lo + ngroups * DG + tt
      pltpu.sync_copy(dst_h.at[pl.ds(cid * CH, CH)], didx[0])
      pltpu.sync_copy(ones_v, acc_s.at[didx[0]], add=True)
      return carry

    lax.fori_loop(0, hi - lo - ngroups * DG, tail, 0)
    plsc.subcore_barrier()

    def wb(off):
      @pl.when(c == 0)
      def _():
        pltpu.sync_copy(acc_s.at[pl.ds(off, RW)], degA_h.at[pl.ds(off, RW)])

      @pl.when(c == 1)
      def _():
        pltpu.sync_copy(acc_s.at[pl.ds(off, RW)], degB_h.at[pl.ds(off, RW)])

    _row_chunks(s, wb)

  return k(dst, ones_h, zrow)


HBCH = 4


def _sc_gather(h, bidx, sidxt):
  @functools.partial(
      pl.kernel,
      out_type=(jax.ShapeDtypeStruct((BB, OUTF), jnp.float32),
                jax.ShapeDtypeStruct((BB * KK, OUTF), jnp.float32)),
      mesh=plsc.VectorSubcoreMesh(**_SC_MESH),
      scratch_types=(
          [pltpu.VMEM((CH,), jnp.int32)] * (HBCH + 1) +
          [pltpu.VMEM((CH, OUTF), jnp.float32)] * 3 +
          [pltpu.SemaphoreType.DMA((HBCH + 1,)),
           pltpu.SemaphoreType.DMA((HBCH + 1,)),
           pltpu.SemaphoreType.DMA((HBCH + 1,))]),
  )
  def k(h_h, bidx_h, sidxt_h, ha_h, hb_h, *refs):
    idxb = refs[:HBCH + 1]
    rows = refs[HBCH + 1:HBCH + 4]
    semi, semg, semw = refs[HBCH + 4:]
    c = lax.axis_index("c")
    s = lax.axis_index("s")
    wid = s * NC + c

    di = []
    for q in range(HBCH):
      di.append(pltpu.async_copy(
          sidxt_h.at[pl.ds(wid * (HBCH * CH) + q * CH, CH)], idxb[q],
          semi.at[q]))
    dg = {}
    dw = {}
    waited = set()
    for q in range(HBCH + 1):
      if q < HBCH:
        b = q % 3
        if q >= 3:
          dw[q - 3].wait()
          waited.add(q - 3)
        di[q].wait()
        dg[q] = pltpu.async_copy(h_h.at[idxb[q]], rows[b], semg.at[q])
      if 1 <= q:
        p = q - 1
        dg[p].wait()
        dw[p] = pltpu.async_copy(
            rows[p % 3], hb_h.at[pl.ds(wid * (HBCH * CH) + p * CH, CH)],
            semw.at[p])
    for q in range(HBCH):
      if q not in waited:
        dw[q].wait()

    @pl.when(wid < BB // CH)
    def _():
      pltpu.sync_copy(bidx_h.at[pl.ds(wid * CH, CH)], idxb[HBCH])
      pltpu.async_copy(h_h.at[idxb[HBCH]], rows[0], semg.at[HBCH]).wait()
      pltpu.sync_copy(rows[0], ha_h.at[pl.ds(wid * CH, CH)])

  return k(h, bidx, sidxt)


RB = 400


def _tc_stage1(x, aggA, aggB, degA, degB, W1, b1, W2, b2):
  def body(x_r, aA_r, aB_r, dA_r, dB_r, W1_r, b1_r, W2_r, b2_r, oA_r, oB_r):
    agg = jnp.concatenate([aA_r[...], aB_r[...]], axis=1)
    rec = 1.0 / (dA_r[...][:, 0:1] + dB_r[...][:, 0:1] + 1.0)
    z1 = (agg + x_r[...]) * rec
    h1 = jnp.dot(z1, W1_r[...], preferred_element_type=jnp.float32) + b1_r[...]
    h1 = jnp.maximum(h1, 0.0)
    y2 = jnp.dot(h1, W2_r[...], preferred_element_type=jnp.float32) + b2_r[...]
    oA_r[...] = y2[:, :HALF]
    oB_r[...] = y2[:, HALF:]

  return pl.pallas_call(
      body,
      grid=(N // RB,),
      in_specs=[
          pl.BlockSpec((RB, D_IN), lambda i: (i, 0)),
          pl.BlockSpec((RB, HALF), lambda i: (i, 0)),
          pl.BlockSpec((RB, HALF), lambda i: (i, 0)),
          pl.BlockSpec((RB, HALF), lambda i: (i, 0)),
          pl.BlockSpec((RB, HALF), lambda i: (i, 0)),
          pl.BlockSpec((D_IN, HID), lambda i: (0, 0)),
          pl.BlockSpec((1, HID), lambda i: (0, 0)),
          pl.BlockSpec((HID, OUTF), lambda i: (0, 0)),
          pl.BlockSpec((1, OUTF), lambda i: (0, 0)),
      ],
      out_specs=[
          pl.BlockSpec((RB, HALF), lambda i: (i, 0)),
          pl.BlockSpec((RB, HALF), lambda i: (i, 0)),
      ],
      out_shape=[
          jax.ShapeDtypeStruct((N, HALF), jnp.float32),
          jax.ShapeDtypeStruct((N, HALF), jnp.float32),
      ],
  )(x, aggA, aggB, degA, degB, W1, b1, W2, b2)


def _tc_stage2(y2A, y2B, aggA, aggB, degA, degB):
  def body(yA_r, yB_r, aA_r, aB_r, dA_r, dB_r, h_r):
    rec = 1.0 / (dA_r[...][:, 0:1] + dB_r[...][:, 0:1] + 1.0)
    left = (aA_r[...] + yA_r[...]) * rec
    right = (aB_r[...] + yB_r[...]) * rec
    h_r[...] = jnp.concatenate([left, right], axis=1)

  return pl.pallas_call(
      body,
      grid=(N // RB,),
      in_specs=[pl.BlockSpec((RB, HALF), lambda i: (i, 0))] * 6,
      out_specs=pl.BlockSpec((RB, OUTF), lambda i: (i, 0)),
      out_shape=jax.ShapeDtypeStruct((N, OUTF), jnp.float32),
  )(y2A, y2B, aggA, aggB, degA, degB)


DBB = 512


def _tc_ha(ha_rows, Wd, bd):
  def body(a_r, Wd_r, bd_r, o_r, n_r):
    had = jnp.dot(a_r[...], Wd_r[...],
                  preferred_element_type=jnp.float32) + bd_r[...]
    o_r[...] = had
    n_r[...] = jnp.sum(had * had, axis=1, keepdims=True)

  return pl.pallas_call(
      body,
      grid=(BB // DBB,),
      in_specs=[
          pl.BlockSpec((DBB, OUTF), lambda i: (i, 0)),
          pl.BlockSpec((OUTF, DECF), lambda i: (0, 0)),
          pl.BlockSpec((1, DECF), lambda i: (0, 0)),
      ],
      out_specs=[
          pl.BlockSpec((DBB, DECF), lambda i: (i, 0)),
          pl.BlockSpec((DBB, 1), lambda i: (i, 0)),
      ],
      out_shape=[
          jax.ShapeDtypeStruct((BB, DECF), jnp.float32),
          jax.ShapeDtypeStruct((BB, 1), jnp.float32),
      ],
  )(ha_rows, Wd, bd)


def _tc_decoder(ha_dec, na2, hb_rows, Wd, bd):
  def body(a_r, n_r, b_r, Wd_r, bd_r, o_r):
    hb = jnp.dot(b_r[...], Wd_r[...],
                 preferred_element_type=jnp.float32) + bd_r[...]
    num = jnp.sum(a_r[...] * hb, axis=1, keepdims=True)
    nb2 = jnp.sum(hb * hb, axis=1, keepdims=True)
    o_r[...] = num / jnp.maximum(jnp.sqrt(n_r[...] * nb2), 1e-8)

  nb = BB // DBB
  return pl.pallas_call(
      body,
      grid=(KK, nb),
      in_specs=[
          pl.BlockSpec((DBB, DECF), lambda k, i: (i, 0)),
          pl.BlockSpec((DBB, 1), lambda k, i: (i, 0)),
          pl.BlockSpec((DBB, OUTF), lambda k, i: (k * (BB // DBB) + i, 0)),
          pl.BlockSpec((OUTF, DECF), lambda k, i: (0, 0)),
          pl.BlockSpec((1, DECF), lambda k, i: (0, 0)),
      ],
      out_specs=pl.BlockSpec((DBB, 1), lambda k, i: (k * (BB // DBB) + i, 0)),
      out_shape=jax.ShapeDtypeStruct((BB * KK, 1), jnp.float32),
  )(ha_dec, na2, hb_rows, Wd, bd)


def kernel(x, edge_index, shuffled_index, sample_batch, W1, b1, W2, b2, Wd, bd):
  src = edge_index[0]
  dst = edge_index[1]
  xA = x[:, :HALF]
  xB = x[:, HALF:]
  zrow = jnp.zeros((RW, HALF), jnp.float32)
  ones_h = jnp.ones((CH, HALF), jnp.float32)

  degA, degB = _sc_deg(dst, ones_h, zrow)
  aggA, aggB = _sc_segsum(xA, xB, src, dst, zrow, HALF)
  y2A, y2B = _tc_stage1(x, aggA, aggB, degA, degB,
                        W1, b1.reshape(1, HID), W2, b2.reshape(1, OUTF))
  agg2A, agg2B = _sc_segsum(y2A, y2B, src, dst, zrow, HALF)
  h = _tc_stage2(y2A, y2B, agg2A, agg2B, degA, degB)

  sidxt = shuffled_index.T.reshape(-1)
  ha_rows, hb_rows = _sc_gather(h, sample_batch, sidxt)
  ha_dec, na2 = _tc_ha(ha_rows, Wd, bd.reshape(1, DECF))
  dec_t = _tc_decoder(ha_dec, na2, hb_rows, Wd, bd.reshape(1, DECF))
  dec = dec_t.reshape(KK, BB).T
  return (h, dec)
